# Initial kernel scaffold; baseline (speedup 1.0000x reference)
import functools

import jax
import jax.numpy as jnp
from jax import lax
from jax.experimental import pallas as pl
from jax.experimental.pallas import tpu as pltpu
from jax.experimental.pallas import tpu_sc as plsc

D = 32
NC, NS = 2, 16
NW = NC * NS
CHUNK = 1024


def _emb_body(ids_hbm, table_hbm, out_hbm, idx_v, rows_v, sem, *, b_per_w):
    wid = lax.axis_index("s") * NC + lax.axis_index("c")
    base = wid * b_per_w
    n_chunks = b_per_w // CHUNK

    def step(i, _):
        off = base + i * CHUNK
        pltpu.sync_copy(ids_hbm.at[pl.ds(off, CHUNK)], idx_v)
        pltpu.async_copy(table_hbm.at[idx_v], rows_v, sem).wait()
        pltpu.sync_copy(rows_v, out_hbm.at[pl.ds(off, CHUNK)])
        return 0

    lax.fori_loop(0, n_chunks, step, 0)


def kernel(token_ids, weights):
    bsz, seq = token_ids.shape
    b_total = bsz * seq
    b_per_w = b_total // NW
    flat_ids = token_ids.reshape(b_total)

    mesh = plsc.VectorSubcoreMesh(
        core_axis_name="c", subcore_axis_name="s", num_cores=NC, num_subcores=NS
    )
    grab = pl.kernel(
        functools.partial(_emb_body, b_per_w=b_per_w),
        out_type=jax.ShapeDtypeStruct((b_total, D), jnp.float32),
        mesh=mesh,
        scratch_types=[
            pltpu.VMEM((CHUNK,), jnp.int32),
            pltpu.VMEM((CHUNK, D), jnp.float32),
            pltpu.SemaphoreType.DMA,
        ],
    )
    out = grab(flat_ids, weights)
    return out.reshape(bsz, seq, D)


# speedup vs baseline: 1.0954x; 1.0954x over previous
import functools

import jax
import jax.numpy as jnp
from jax import lax
from jax.experimental import pallas as pl
from jax.experimental.pallas import tpu as pltpu
from jax.experimental.pallas import tpu_sc as plsc

D = 32
NC, NS = 2, 16
NW = NC * NS
CHUNK = 1024


def _emb_body(ids_hbm, table_hbm, out_hbm, idx_v, rows_v, sem, *, b_per_w):
    wid = lax.axis_index("s") * NC + lax.axis_index("c")
    base = wid * b_per_w
    n_chunks = b_per_w // CHUNK

    def step(i, _):
        off = base + i * CHUNK
        pltpu.sync_copy(ids_hbm.at[pl.ds(off, CHUNK)], idx_v)
        pltpu.async_copy(table_hbm.at[idx_v], rows_v, sem).wait()
        pltpu.sync_copy(rows_v, out_hbm.at[pl.ds(off, CHUNK)])
        return 0

    lax.fori_loop(0, n_chunks, step, 0)


def kernel(token_ids, weights):
    bsz, seq = token_ids.shape
    b_total = bsz * seq
    b_per_w = b_total // NW
    flat_ids = token_ids.reshape(b_total)

    mesh = plsc.VectorSubcoreMesh(
        core_axis_name="c", subcore_axis_name="s", num_cores=NC, num_subcores=NS
    )
    grab = pl.kernel(
        functools.partial(_emb_body, b_per_w=b_per_w),
        out_type=jax.ShapeDtypeStruct((b_total, D), jnp.float32),
        mesh=mesh,
        scratch_types=[
            pltpu.VMEM((CHUNK,), jnp.int32),
            pltpu.VMEM((CHUNK, D), jnp.float32),
            pltpu.SemaphoreType.DMA,
        ],
        compiler_params=pltpu.CompilerParams(use_tc_tiling_on_sc=False),
    )
    out = grab(flat_ids, weights)
    return out.reshape(bsz, seq, D)


# speedup vs baseline: 1.1132x; 1.0163x over previous
import functools

import jax
import jax.numpy as jnp
from jax import lax
from jax.experimental import pallas as pl
from jax.experimental.pallas import tpu as pltpu
from jax.experimental.pallas import tpu_sc as plsc

D = 32
NC, NS = 2, 16
NW = NC * NS
CHUNK = 640
NBUF = 4


def _emb_body(ids_hbm, table_hbm, out_hbm, idx_v, rows_v, isem, gsems, ssems,
              *, b_per_w):
    wid = lax.axis_index("s") * NC + lax.axis_index("c")
    base = wid * b_per_w
    n_chunks = b_per_w // CHUNK
    n_groups = n_chunks // NBUF

    pltpu.async_copy(ids_hbm.at[pl.ds(base, b_per_w)], idx_v, isem).wait()

    def gather(i, b):
        return pltpu.async_copy(
            table_hbm.at[idx_v.at[pl.ds(i * CHUNK, CHUNK)]],
            rows_v.at[b], gsems[b])

    def store(i, b):
        return pltpu.async_copy(
            rows_v.at[b], out_hbm.at[pl.ds(base + i * CHUNK, CHUNK)], ssems[b])

    def wait_gather(b):
        pltpu.make_async_copy(
            table_hbm.at[idx_v.at[pl.ds(0, CHUNK)]], rows_v.at[b],
            gsems[b]).wait()

    def wait_store(b):
        pltpu.make_async_copy(
            rows_v.at[b], out_hbm.at[pl.ds(base, CHUNK)], ssems[b]).wait()

    for b in range(NBUF):
        gather(b, b)

    def group(j, _):
        for b in range(NBUF):
            i = j * NBUF + b
            wait_gather(b)
            store(i, b)

            @pl.when(j < n_groups - 1)
            def _():
                wait_store(b)
                gather(i + NBUF, b)
        return 0

    lax.fori_loop(0, n_groups, group, 0)
    for b in range(NBUF):
        wait_store(b)


def kernel(token_ids, weights):
    bsz, seq = token_ids.shape
    b_total = bsz * seq
    b_per_w = b_total // NW
    flat_ids = token_ids.reshape(b_total)

    mesh = plsc.VectorSubcoreMesh(
        core_axis_name="c", subcore_axis_name="s", num_cores=NC, num_subcores=NS
    )
    grab = pl.kernel(
        functools.partial(_emb_body, b_per_w=b_per_w),
        out_type=jax.ShapeDtypeStruct((b_total, D), jnp.float32),
        mesh=mesh,
        scratch_types=[
            pltpu.VMEM((b_per_w,), jnp.int32),
            pltpu.VMEM((NBUF, CHUNK, D), jnp.float32),
            pltpu.SemaphoreType.DMA,
            [pltpu.SemaphoreType.DMA] * NBUF,
            [pltpu.SemaphoreType.DMA] * NBUF,
        ],
        compiler_params=pltpu.CompilerParams(use_tc_tiling_on_sc=False),
    )
    out = grab(flat_ids, weights)
    return out.reshape(bsz, seq, D)


# speedup vs baseline: 1.6065x; 1.4431x over previous
import functools

import jax
import jax.numpy as jnp
from jax import lax
from jax.experimental import pallas as pl
from jax.experimental.pallas import tpu as pltpu
from jax.experimental.pallas import tpu_sc as plsc

D = 32
NC, NS = 2, 16
NW = NC * NS
SEQ = 50
BATCH = 16384
TOK = 512
NBLK = SEQ * (BATCH // TOK)
PER_W = NBLK // NW
NBUF = 2


def _emb_body(tok_hbm, table_hbm, out_hbm, idx_v, buf_v, tbuf_v,
              isems, gsems, ssems):
    wid = lax.axis_index("s") * NC + lax.axis_index("c")
    lane = lax.iota(jnp.int32, 16)

    def sb(m):
        return m >> 5, m & 31

    def load_ids(m, b):
        s, b4 = sb(m)
        return pltpu.async_copy(
            tok_hbm.at[s, pl.ds(b4 * TOK, TOK)], idx_v.at[b], isems[b])

    def gather(b):
        return pltpu.async_copy(
            table_hbm.at[idx_v.at[b]], buf_v.at[b], gsems[b])

    def wait_ids(b):
        pltpu.make_async_copy(
            tok_hbm.at[0, pl.ds(0, TOK)], idx_v.at[b], isems[b]).wait()

    def wait_gather(b):
        pltpu.make_async_copy(
            table_hbm.at[idx_v.at[b]], buf_v.at[b], gsems[b]).wait()

    def wait_stores(b):
        for k in range(4):
            pltpu.make_async_copy(
                tbuf_v.at[b, k], out_hbm.at[0, k, pl.ds(0, 4)],
                ssems[b]).wait()

    def transpose_and_store(m, b):
        s, b4 = sb(m)

        def col_group(kq, _):
            k = kq >> 2
            bq = kq & 3
            cbase = bq * 128
            for r in range(8):
                dim = k * 8 + r
                cols = dim + jnp.zeros((16,), jnp.int32)
                for c16 in range(8):
                    rows = cbase + c16 * 16 + lane
                    vals = plsc.load_gather(buf_v.at[b], [rows, cols])
                    tbuf_v[b, k, bq, r, pl.ds(c16 * 16, 16)] = vals
            return 0

        lax.fori_loop(0, 16, col_group, 0)
        for k in range(4):
            pltpu.async_copy(
                tbuf_v.at[b, k], out_hbm.at[s, k, pl.ds(b4 * 4, 4)],
                ssems[b])

    m0 = wid * PER_W
    load_ids(m0, 0)
    wait_ids(0)
    gather(0)
    load_ids(m0 + 1, 1)

    def group(j, _):
        for b in (0, 1):
            i = j * 2 + b
            nb = 1 - b
            wait_gather(b)

            @pl.when(i < PER_W - 1)
            def _():
                wait_ids(nb)
                gather(nb)

            @pl.when(i >= 2)
            def _():
                wait_stores(b)

            transpose_and_store(m0 + i, b)

            @pl.when(i < PER_W - 2)
            def _():
                load_ids(m0 + i + 2, b)
        return 0

    lax.fori_loop(0, PER_W // 2, group, 0)
    wait_stores(0)
    wait_stores(1)


def kernel(token_ids, weights):
    tok_t = token_ids.T

    mesh = plsc.VectorSubcoreMesh(
        core_axis_name="c", subcore_axis_name="s", num_cores=NC, num_subcores=NS
    )
    grab = pl.kernel(
        _emb_body,
        out_type=jax.ShapeDtypeStruct((SEQ, 4, BATCH // 128, 8, 128),
                                      jnp.float32),
        mesh=mesh,
        scratch_types=[
            pltpu.VMEM((NBUF, TOK), jnp.int32),
            pltpu.VMEM((NBUF, TOK, D), jnp.float32),
            pltpu.VMEM((NBUF, 4, 4, 8, 128), jnp.float32),
            [pltpu.SemaphoreType.DMA] * NBUF,
            [pltpu.SemaphoreType.DMA] * NBUF,
            [pltpu.SemaphoreType.DMA] * NBUF,
        ],
        compiler_params=pltpu.CompilerParams(
            use_tc_tiling_on_sc=False, needs_layout_passes=False),
    )
    out5 = grab(tok_t, weights)
    return out5.transpose(2, 4, 0, 1, 3).reshape(BATCH, SEQ, D)


# speedup vs baseline: 1.7878x; 1.1129x over previous
import jax
import jax.numpy as jnp
from jax import lax
from jax.experimental import pallas as pl
from jax.experimental.pallas import tpu as pltpu
from jax.experimental.pallas import tpu_sc as plsc

D = 32
NC, NS = 2, 16
NW = NC * NS
SEQ = 50
BATCH = 16384
TOK = 512
NBLK = SEQ * (BATCH // TOK)
PER_W = NBLK // NW
NBUF = 2
BLK_WORDS = TOK * D
OUT_WORDS = SEQ * 4 * (BATCH // 128) * 8 * 128


def _emb_body(tok_hbm, table_hbm, out_hbm, idx_v, buf_v, tbuf_v,
              isems, gsems, ssems):
    wid = lax.axis_index("s") * NC + lax.axis_index("c")
    lane = lax.iota(jnp.int32, 16)
    p0 = ((lane >> 3) << 12) + ((lane & 7) << 7)
    p1 = p0 + 8192

    def sb(m):
        return m >> 5, m & 31

    def load_ids(m, b):
        s, b4 = sb(m)
        return pltpu.async_copy(
            tok_hbm.at[s, pl.ds(b4 * TOK, TOK)], idx_v.at[b], isems[b])

    def gather(b):
        return pltpu.async_copy(
            table_hbm.at[idx_v.at[b]], buf_v.at[b], gsems[b])

    def wait_ids(b):
        pltpu.make_async_copy(
            tok_hbm.at[0, pl.ds(0, TOK)], idx_v.at[b], isems[b]).wait()

    def wait_gather(b):
        pltpu.make_async_copy(
            table_hbm.at[idx_v.at[b]], buf_v.at[b], gsems[b]).wait()

    def wait_stores(b):
        for k in range(4):
            pltpu.make_async_copy(
                tbuf_v.at[b, pl.ds(k * 4096, 4096)],
                out_hbm.at[pl.ds(0, 4096)], ssems[b]).wait()

    def transpose_and_store(m, b):
        s, b4 = sb(m)

        def tok_group(g, _):
            for u in range(8):
                c = g * 8 + u
                base = ((c >> 7) << 10) + (c & 127)
                v0 = buf_v[b, c, pl.ds(0, 16)]
                plsc.store_scatter(tbuf_v.at[b], [p0 + base], v0)
                v1 = buf_v[b, c, pl.ds(16, 16)]
                plsc.store_scatter(tbuf_v.at[b], [p1 + base], v1)
            return 0

        lax.fori_loop(0, TOK // 8, tok_group, 0)
        out0 = (s * 4 * 128 + b4 * 4) * 1024
        for k in range(4):
            pltpu.async_copy(
                tbuf_v.at[b, pl.ds(k * 4096, 4096)],
                out_hbm.at[pl.ds(out0 + k * 131072, 4096)], ssems[b])

    m0 = wid * PER_W
    load_ids(m0, 0)
    wait_ids(0)
    gather(0)
    load_ids(m0 + 1, 1)

    def group(j, _):
        for b in (0, 1):
            i = j * 2 + b
            nb = 1 - b
            wait_gather(b)

            @pl.when(i < PER_W - 1)
            def _():
                wait_ids(nb)
                gather(nb)

            @pl.when(i >= 2)
            def _():
                wait_stores(b)

            transpose_and_store(m0 + i, b)

            @pl.when(i < PER_W - 2)
            def _():
                load_ids(m0 + i + 2, b)
        return 0

    lax.fori_loop(0, PER_W // 2, group, 0)
    wait_stores(0)
    wait_stores(1)


def kernel(token_ids, weights):
    tok_t = token_ids.T

    mesh = plsc.VectorSubcoreMesh(
        core_axis_name="c", subcore_axis_name="s", num_cores=NC, num_subcores=NS
    )
    grab = pl.kernel(
        _emb_body,
        out_type=jax.ShapeDtypeStruct((OUT_WORDS,), jnp.float32),
        mesh=mesh,
        scratch_types=[
            pltpu.VMEM((NBUF, TOK), jnp.int32),
            pltpu.VMEM((NBUF, TOK, D), jnp.float32),
            pltpu.VMEM((NBUF, 4 * 4096), jnp.float32),
            [pltpu.SemaphoreType.DMA] * NBUF,
            [pltpu.SemaphoreType.DMA] * NBUF,
            [pltpu.SemaphoreType.DMA] * NBUF,
        ],
        compiler_params=pltpu.CompilerParams(
            use_tc_tiling_on_sc=False, needs_layout_passes=False),
    )
    out1 = grab(tok_t, weights)
    out5 = out1.reshape(SEQ, 4, BATCH // 128, 8, 128)
    return out5.transpose(2, 4, 0, 1, 3).reshape(BATCH, SEQ, D)


# speedup vs baseline: 1.9257x; 1.0771x over previous
import jax
import jax.numpy as jnp
from jax import lax
from jax.experimental import pallas as pl
from jax.experimental.pallas import tpu as pltpu
from jax.experimental.pallas import tpu_sc as plsc

D = 32
NC, NS = 2, 16
NW = NC * NS
SEQ = 50
BATCH = 16384
TOK = 512
NBLK = SEQ * (BATCH // TOK)
PER_W = NBLK // NW
NBUF = 2
BLK_WORDS = TOK * D
OUT_WORDS = SEQ * 4 * (BATCH // 128) * 8 * 128


def _emb_body(tok_hbm, table_hbm, out_hbm, idx_v, buf_v, tbuf_v,
              isems, gsems, ssems):
    wid = lax.axis_index("s") * NC + lax.axis_index("c")
    lane = lax.iota(jnp.int32, 16)
    lane32 = lane << 5

    def sb(m):
        return m >> 5, m & 31

    def load_ids(m, b):
        s, b4 = sb(m)
        return pltpu.async_copy(
            tok_hbm.at[s, pl.ds(b4 * TOK, TOK)], idx_v.at[b], isems[b])

    def gather(b):
        return pltpu.async_copy(
            table_hbm.at[idx_v.at[b]], buf_v.at[b], gsems[b])

    def wait_ids(b):
        pltpu.make_async_copy(
            tok_hbm.at[0, pl.ds(0, TOK)], idx_v.at[b], isems[b]).wait()

    def wait_gather(b):
        pltpu.make_async_copy(
            table_hbm.at[idx_v.at[b]], buf_v.at[b], gsems[b]).wait()

    def wait_stores(b):
        for k in range(4):
            pltpu.make_async_copy(
                tbuf_v.at[b, pl.ds(k * 4096, 4096)],
                out_hbm.at[pl.ds(0, 4096)], ssems[b]).wait()

    def transpose_and_store(m, b):
        s, b4 = sb(m)

        @plsc.parallel_loop(0, 32, unroll=4)
        def _(t):
            bq = t >> 3
            c16 = t & 7
            rows = (bq << 7) + (c16 << 4) + lane
            dstbase = (bq << 10) + (c16 << 4)
            for d in range(32):
                dst = (d >> 3) * 4096 + (d & 7) * 128
                cols = jnp.zeros((16,), jnp.int32) + d
                vals = plsc.load_gather(buf_v.at[b], [rows, cols])
                tbuf_v[b, pl.ds(dstbase + dst, 16)] = vals
        out0 = (s * 4 * 128 + b4 * 4) * 1024
        for k in range(4):
            pltpu.async_copy(
                tbuf_v.at[b, pl.ds(k * 4096, 4096)],
                out_hbm.at[pl.ds(out0 + k * 131072, 4096)], ssems[b])

    m0 = wid * PER_W
    load_ids(m0, 0)
    wait_ids(0)
    gather(0)
    load_ids(m0 + 1, 1)

    def group(j, _):
        for b in (0, 1):
            i = j * 2 + b
            nb = 1 - b
            wait_gather(b)

            @pl.when(i < PER_W - 1)
            def _():
                wait_ids(nb)
                gather(nb)

            @pl.when(i >= 2)
            def _():
                wait_stores(b)

            transpose_and_store(m0 + i, b)

            @pl.when(i < PER_W - 2)
            def _():
                load_ids(m0 + i + 2, b)
        return 0

    lax.fori_loop(0, PER_W // 2, group, 0)
    wait_stores(0)
    wait_stores(1)


def kernel(token_ids, weights):
    tok_t = token_ids.T

    mesh = plsc.VectorSubcoreMesh(
        core_axis_name="c", subcore_axis_name="s", num_cores=NC, num_subcores=NS
    )
    grab = pl.kernel(
        _emb_body,
        out_type=jax.ShapeDtypeStruct((OUT_WORDS,), jnp.float32),
        mesh=mesh,
        scratch_types=[
            pltpu.VMEM((NBUF, TOK), jnp.int32),
            pltpu.VMEM((NBUF, TOK, D), jnp.float32),
            pltpu.VMEM((NBUF, 4 * 4096), jnp.float32),
            [pltpu.SemaphoreType.DMA] * NBUF,
            [pltpu.SemaphoreType.DMA] * NBUF,
            [pltpu.SemaphoreType.DMA] * NBUF,
        ],
        compiler_params=pltpu.CompilerParams(
            use_tc_tiling_on_sc=False, needs_layout_passes=False),
    )
    out1 = grab(tok_t, weights)
    out5 = out1.reshape(SEQ, 4, BATCH // 128, 8, 128)
    return out5.transpose(2, 4, 0, 1, 3).reshape(BATCH, SEQ, D)


# speedup vs baseline: 2.0768x; 1.0785x over previous
import jax
import jax.numpy as jnp
from jax import lax
from jax.experimental import pallas as pl
from jax.experimental.pallas import tpu as pltpu
from jax.experimental.pallas import tpu_sc as plsc

D = 32
NC, NS = 2, 16
NW = NC * NS
SEQ = 50
BATCH = 16384
TOK = 512
NBLK = SEQ * (BATCH // TOK)
PER_W = NBLK // NW
NBUF = 2
BLK_WORDS = TOK * D
OUT_WORDS = SEQ * 4 * (BATCH // 128) * 8 * 128


def _emb_body(tok_hbm, table_hbm, out_hbm, idx_v, buf_v, tbuf_v,
              isems, gsems, ssems):
    wid = lax.axis_index("s") * NC + lax.axis_index("c")
    lane = lax.iota(jnp.int32, 16)
    p0 = ((lane >> 3) << 12) + ((lane & 7) << 7)
    p1 = p0 + 8192

    def sb(m):
        return m >> 5, m & 31

    def load_ids(m, b):
        s, b4 = sb(m)
        return pltpu.async_copy(
            tok_hbm.at[s, pl.ds(b4 * TOK, TOK)], idx_v.at[b], isems[b])

    def gather(b):
        return pltpu.async_copy(
            table_hbm.at[idx_v.at[b]], buf_v.at[b], gsems[b])

    def wait_ids(b):
        pltpu.make_async_copy(
            tok_hbm.at[0, pl.ds(0, TOK)], idx_v.at[b], isems[b]).wait()

    def wait_gather(b):
        pltpu.make_async_copy(
            table_hbm.at[idx_v.at[b]], buf_v.at[b], gsems[b]).wait()

    def wait_stores(b):
        for k in range(4):
            pltpu.make_async_copy(
                tbuf_v.at[b, pl.ds(k * 4096, 4096)],
                out_hbm.at[pl.ds(0, 4096)], ssems[b]).wait()

    def transpose_and_store(m, b):
        s, b4 = sb(m)

        for bq in range(4):
            q0 = p0 + (bq << 10)
            q1 = p1 + (bq << 10)
            cb = bq << 7

            @plsc.parallel_loop(0, 128, unroll=8)
            def _(cc, q0=q0, q1=q1, cb=cb):
                c = cb + cc
                v0 = buf_v[b, c, pl.ds(0, 16)]
                plsc.store_scatter(tbuf_v.at[b], [q0 + cc], v0)
                v1 = buf_v[b, c, pl.ds(16, 16)]
                plsc.store_scatter(tbuf_v.at[b], [q1 + cc], v1)
        out0 = (s * 4 * 128 + b4 * 4) * 1024
        for k in range(4):
            pltpu.async_copy(
                tbuf_v.at[b, pl.ds(k * 4096, 4096)],
                out_hbm.at[pl.ds(out0 + k * 131072, 4096)], ssems[b])

    m0 = wid * PER_W
    load_ids(m0, 0)
    wait_ids(0)
    gather(0)
    load_ids(m0 + 1, 1)

    def group(j, _):
        for b in (0, 1):
            i = j * 2 + b
            nb = 1 - b
            wait_gather(b)

            @pl.when(i < PER_W - 1)
            def _():
                wait_ids(nb)
                gather(nb)

            @pl.when(i >= 2)
            def _():
                wait_stores(b)

            transpose_and_store(m0 + i, b)

            @pl.when(i < PER_W - 2)
            def _():
                load_ids(m0 + i + 2, b)
        return 0

    lax.fori_loop(0, PER_W // 2, group, 0)
    wait_stores(0)
    wait_stores(1)


def kernel(token_ids, weights):
    tok_t = token_ids.T

    mesh = plsc.VectorSubcoreMesh(
        core_axis_name="c", subcore_axis_name="s", num_cores=NC, num_subcores=NS
    )
    grab = pl.kernel(
        _emb_body,
        out_type=jax.ShapeDtypeStruct((OUT_WORDS,), jnp.float32),
        mesh=mesh,
        scratch_types=[
            pltpu.VMEM((NBUF, TOK), jnp.int32),
            pltpu.VMEM((NBUF, TOK, D), jnp.float32),
            pltpu.VMEM((NBUF, 4 * 4096), jnp.float32),
            [pltpu.SemaphoreType.DMA] * NBUF,
            [pltpu.SemaphoreType.DMA] * NBUF,
            [pltpu.SemaphoreType.DMA] * NBUF,
        ],
        compiler_params=pltpu.CompilerParams(
            use_tc_tiling_on_sc=False, needs_layout_passes=False),
    )
    out1 = grab(tok_t, weights)
    out5 = out1.reshape(SEQ, 4, BATCH // 128, 8, 128)
    return out5.transpose(2, 4, 0, 1, 3).reshape(BATCH, SEQ, D)
